# baseline (speedup 1.0000x reference)
import functools

import jax
import jax.numpy as jnp
from jax import lax
from jax.experimental import pallas as pl
from jax.experimental.pallas import tpu as pltpu
from jax.experimental.pallas import tpu_sc as plsc

NUM_EMB = 1_000_000
DIM = 128
NCB = 8
CBS = 256
SUB = 16
BATCH = 16384

_INFO = plsc.get_sparse_core_info()
NC, NS, L = _INFO.num_cores, _INFO.num_subcores, _INFO.num_lanes
NW = NC * NS
BPW = BATCH // NW
CHUNK = 128
NIC = BPW // CHUNK
NGC = BPW * NCB // CHUNK

PCG = 512
PSTEPS = 61
PPW = PCG * PSTEPS
PMAIN = PPW * NW
PTAIL = NUM_EMB - PMAIN


def _pack16(in_ref, src, lo_ref, hi_ref, dst):
    w = [in_ref[i, src] for i in range(NCB)]
    lo_ref[dst] = w[0] | (w[1] << 8) | (w[2] << 16) | (w[3] << 24)
    hi_ref[dst] = w[4] | (w[5] << 8) | (w[6] << 16) | (w[7] << 24)


NBUF = 6


def _pack_body(ct_hbm, tail_hbm, lo_hbm, hi_hbm, *bufrefs):
    ins, los, his = (bufrefs[:NBUF], bufrefs[NBUF:2 * NBUF],
                     bufrefs[2 * NBUF:3 * NBUF])
    sem_in, sem_out = bufrefs[3 * NBUF:]
    in0_v, lo0_v, hi0_v = ins[0], los[0], his[0]
    wid = lax.axis_index("s") * NC + lax.axis_index("c")
    w0 = wid * PPW
    bufs = tuple(zip(ins, los, his))

    for b in range(NBUF):
        pltpu.async_copy(ct_hbm.at[:, pl.ds(w0 + b * PCG, PCG)],
                         bufs[b][0], sem_in)

    def emit_chunk(c, buf):
        in_v, lo_v, hi_v = bufs[buf]
        e0 = w0 + c * PCG
        pltpu.make_async_copy(ct_hbm.at[:, pl.ds(e0, PCG)],
                              in_v, sem_in).wait()

        @pl.when(c >= NBUF)
        def _():
            o0 = w0 + (c - NBUF) * PCG
            pltpu.make_async_copy(lo_v, lo_hbm.at[pl.ds(o0, PCG)],
                                  sem_out).wait()
            pltpu.make_async_copy(hi_v, hi_hbm.at[pl.ds(o0, PCG)],
                                  sem_out).wait()

        for k in range(PCG // L):
            _pack16(in_v, pl.ds(k * L, L), lo_v, hi_v, pl.ds(k * L, L))

        @pl.when(c + NBUF < PSTEPS)
        def _():
            pltpu.async_copy(ct_hbm.at[:, pl.ds(e0 + NBUF * PCG, PCG)],
                             in_v, sem_in)

        pltpu.async_copy(lo_v, lo_hbm.at[pl.ds(e0, PCG)], sem_out)
        pltpu.async_copy(hi_v, hi_hbm.at[pl.ds(e0, PCG)], sem_out)

    def step(d, carry):
        for b in range(NBUF):
            emit_chunk(NBUF * d + b, b)
        return carry

    lax.fori_loop(0, PSTEPS // NBUF, step, 0)
    emit_chunk(PSTEPS - 1, (PSTEPS - 1) % NBUF)

    for c in range(PSTEPS - NBUF, PSTEPS):
        _, lo_v, hi_v = bufs[c % NBUF]
        o0 = w0 + c * PCG
        pltpu.make_async_copy(lo_v, lo_hbm.at[pl.ds(o0, PCG)],
                              sem_out).wait()
        pltpu.make_async_copy(hi_v, hi_hbm.at[pl.ds(o0, PCG)],
                              sem_out).wait()

    @pl.when(wid == NW - 1)
    def _():
        for off, sz, out_sz in ((0, PCG, PCG), (PCG, 128, PTAIL - PCG)):
            pltpu.sync_copy(tail_hbm.at[:, pl.ds(off, sz)],
                            in0_v.at[:, pl.ds(0, sz)])
            for k in range(sz // L):
                _pack16(in0_v, pl.ds(k * L, L), lo0_v, hi0_v,
                        pl.ds(k * L, L))
            pltpu.sync_copy(lo0_v.at[pl.ds(0, out_sz)],
                            lo_hbm.at[pl.ds(PMAIN + off, out_sz)])
            pltpu.sync_copy(hi0_v.at[pl.ds(0, out_sz)],
                            hi_hbm.at[pl.ds(PMAIN + off, out_sz)])


_pack = functools.partial(
    pl.kernel,
    out_type=(jax.ShapeDtypeStruct((NUM_EMB,), jnp.int32),
              jax.ShapeDtypeStruct((NUM_EMB,), jnp.int32)),
    mesh=plsc.VectorSubcoreMesh(core_axis_name="c", subcore_axis_name="s"),
    compiler_params=pltpu.CompilerParams(use_tc_tiling_on_sc=True),
    scratch_types=(
        [pltpu.VMEM((NCB, PCG), jnp.int32)] * NBUF
        + [pltpu.VMEM((PCG,), jnp.int32)] * (2 * NBUF)
        + [pltpu.SemaphoreType.DMA, pltpu.SemaphoreType.DMA]
    ),
)(_pack_body)


def _pq_body(idx_hbm, cb_hbm, lo_hbm, hi_hbm, out_hbm,
             idx_v, lo_v, hi_v, g_v, rows_v, sem, sem2, out_sem):
    wid = lax.axis_index("s") * NC + lax.axis_index("c")

    pltpu.sync_copy(idx_hbm.at[pl.ds(wid * NIC, NIC)], idx_v)

    for c in range(NIC):
        pltpu.async_copy(lo_hbm.at[idx_v.at[c]], lo_v.at[c], sem)
        pltpu.async_copy(hi_hbm.at[idx_v.at[c]], hi_v.at[c], sem)

    for c in range(NIC):
        pltpu.make_async_copy(lo_hbm.at[idx_v.at[c]], lo_v.at[c],
                              sem).wait()
        pltpu.make_async_copy(hi_hbm.at[idx_v.at[c]], hi_v.at[c],
                              sem).wait()
        for i in range(NCB):
            src = lo_v if i < 4 else hi_v
            sh = (i & 3) * 8
            for l in range(CHUNK // L):
                w = src[c, pl.ds(l * L, L)]
                code = lax.shift_right_logical(w, sh) & 255
                g_v[i * NIC + c, pl.ds(l * L, L)] = code + i * CBS
        for i in range(NCB):
            pltpu.async_copy(cb_hbm.at[g_v.at[i * NIC + c]],
                             rows_v.at[i, pl.ds(c * CHUNK, CHUNK)], sem2)

    for c in range(NIC):
        for i in range(NCB):
            pltpu.make_async_copy(
                cb_hbm.at[g_v.at[i * NIC + c]],
                rows_v.at[i, pl.ds(c * CHUNK, CHUNK)], sem2).wait()
    base = wid * BPW
    out_handles = []
    for i in range(NCB):
        out_handles.append(pltpu.async_copy(
            rows_v.at[i],
            out_hbm.at[pl.ds(base, BPW), pl.ds(i * SUB, SUB)], out_sem))
    for h in out_handles:
        h.wait()


_pq_decode = functools.partial(
    pl.kernel,
    out_type=jax.ShapeDtypeStruct((BATCH, DIM), jnp.float32),
    mesh=plsc.VectorSubcoreMesh(core_axis_name="c", subcore_axis_name="s"),
    compiler_params=pltpu.CompilerParams(needs_layout_passes=False,
                                         use_tc_tiling_on_sc=False),
    scratch_types=[
        pltpu.VMEM((NIC, CHUNK), jnp.int32),
        pltpu.VMEM((NIC, CHUNK), jnp.int32),
        pltpu.VMEM((NIC, CHUNK), jnp.int32),
        pltpu.VMEM((NGC, CHUNK), jnp.int32),
        pltpu.VMEM((NCB, BPW, SUB), jnp.float32),
        pltpu.SemaphoreType.DMA,
        pltpu.SemaphoreType.DMA,
        pltpu.SemaphoreType.DMA,
    ],
)(_pq_body)


def kernel(indices, codebooks, codes):
    idx2 = indices.astype(jnp.int32).reshape(BATCH // CHUNK, CHUNK)
    cb_flat = codebooks.reshape(NCB * CBS, SUB)
    tail = jnp.pad(codes[PMAIN:, :].T, ((0, 0), (0, PCG + 128 - PTAIL)))
    lo, hi = _pack(codes.T, tail)
    return _pq_decode(idx2, cb_flat, lo, hi)
